# baseline (speedup 1.0000x reference)
import functools

import jax
import jax.numpy as jnp
from jax.experimental import pallas as pl
from jax.experimental.pallas import tpu as pltpu

V = 1000
D = 128
EPS = 0.2
T_INV = 2.0
BLK = 256


def _s_table_body(emb_ref, s_ref):
    emb = emb_ref[...]
    ss = jnp.sum(emb * emb, axis=1, keepdims=True)
    nrm = jnp.maximum(jnp.sqrt(ss), 1e-12)
    emb_n = emb / nrm
    sim = jax.lax.dot_general(
        emb_n, emb_n, (((1,), (1,)), ((), ())),
        preferred_element_type=jnp.float32)
    rows = jax.lax.broadcasted_iota(jnp.int32, (V, V), 0)
    cols = jax.lax.broadcasted_iota(jnp.int32, (V, V), 1)
    diag = rows == cols
    masked = jnp.where(diag, -jnp.inf, sim * T_INV)
    m = jnp.max(masked, axis=1, keepdims=True)
    e = jnp.exp(masked - m)
    denom = jnp.sum(e, axis=1, keepdims=True)
    s = e * (EPS / denom)
    s_ref[...] = jnp.where(diag, 1.0 - EPS, s).astype(jnp.bfloat16)


def _main_body(logits_ref, labels_ref, s_ref, out_ref, acc_scr):
    i = pl.program_id(0)

    @pl.when(i == 0)
    def _init():
        acc_scr[0] = 0.0

    x = logits_ref[...]
    m = jnp.max(x, axis=1, keepdims=True)
    e = jnp.exp(x - m)
    sm = jnp.sum(e, axis=1)
    lse = jnp.log(sm) + m[:, 0]

    lab = labels_ref[0, 0, :]
    oh = (jax.lax.broadcasted_iota(jnp.int32, (BLK, V), 1)
          == lab[:, None]).astype(jnp.bfloat16)
    sg = jax.lax.dot_general(
        oh, s_ref[...], (((1,), (0,)), ((), ())),
        preferred_element_type=jnp.float32)
    dots = jnp.sum(sg * x)
    acc_scr[0] += jnp.sum(lse) - dots

    @pl.when(i == pl.num_programs(0) - 1)
    def _fin():
        out_ref[0] = acc_scr[0]


def kernel(logits, labels, word_emb_tab):
    logits = logits.astype(jnp.float32)
    labels = labels.astype(jnp.int32)
    batch = logits.shape[0]
    nblk = batch // BLK

    s_tab = pl.pallas_call(
        _s_table_body,
        out_shape=jax.ShapeDtypeStruct((V, V), jnp.bfloat16),
    )(word_emb_tab.astype(jnp.float32))

    labels3 = labels.reshape(nblk, 1, BLK)
    total = pl.pallas_call(
        _main_body,
        grid=(nblk,),
        in_specs=[
            pl.BlockSpec((BLK, V), lambda i: (i, 0)),
            pl.BlockSpec((1, 1, BLK), lambda i: (i, 0, 0)),
            pl.BlockSpec((V, V), lambda i: (0, 0)),
        ],
        out_specs=pl.BlockSpec(memory_space=pltpu.SMEM),
        out_shape=jax.ShapeDtypeStruct((1,), jnp.float32),
        scratch_shapes=[
            pltpu.SMEM((1,), jnp.float32),
        ],
    )(logits, labels3, s_tab)

    return (total[0] / batch).astype(jnp.float32)


# speedup vs baseline: 1.0708x; 1.0708x over previous
import functools

import jax
import jax.numpy as jnp
from jax.experimental import pallas as pl
from jax.experimental.pallas import tpu as pltpu

V = 1000
D = 128
EPS = 0.2
T_INV = 2.0
BLK = 512


def _s_table_body(emb_ref, s_ref):
    emb = emb_ref[...]
    ss = jnp.sum(emb * emb, axis=1, keepdims=True)
    nrm = jnp.maximum(jnp.sqrt(ss), 1e-12)
    emb_n = emb / nrm
    sim = jax.lax.dot_general(
        emb_n, emb_n, (((1,), (1,)), ((), ())),
        preferred_element_type=jnp.float32)
    rows = jax.lax.broadcasted_iota(jnp.int32, (V, V), 0)
    cols = jax.lax.broadcasted_iota(jnp.int32, (V, V), 1)
    diag = rows == cols
    masked = jnp.where(diag, -jnp.inf, sim * T_INV)
    m = jnp.max(masked, axis=1, keepdims=True)
    e = jnp.exp(masked - m)
    denom = jnp.sum(e, axis=1, keepdims=True)
    s = e * (EPS / denom)
    s_ref[...] = jnp.where(diag, 1.0 - EPS, s).astype(jnp.bfloat16)


def _main_body(logits_ref, labels_ref, s_ref, out_ref, acc_scr):
    i = pl.program_id(0)

    @pl.when(i == 0)
    def _init():
        acc_scr[0] = 0.0

    x = logits_ref[...]
    m = jnp.max(x, axis=1, keepdims=True)
    e = jnp.exp(x - m)
    sm = jnp.sum(e, axis=1)
    lse = jnp.log(sm) + m[:, 0]

    lab = labels_ref[0, 0, :]
    oh = (jax.lax.broadcasted_iota(jnp.int32, (BLK, V), 1)
          == lab[:, None]).astype(jnp.bfloat16)
    sg = jax.lax.dot_general(
        oh, s_ref[...], (((1,), (0,)), ((), ())),
        preferred_element_type=jnp.float32)
    dots = jnp.sum(sg * x)
    acc_scr[0] += jnp.sum(lse) - dots

    @pl.when(i == pl.num_programs(0) - 1)
    def _fin():
        out_ref[0] = acc_scr[0]


def kernel(logits, labels, word_emb_tab):
    logits = logits.astype(jnp.float32)
    labels = labels.astype(jnp.int32)
    batch = logits.shape[0]
    nblk = batch // BLK

    s_tab = pl.pallas_call(
        _s_table_body,
        out_shape=jax.ShapeDtypeStruct((V, V), jnp.bfloat16),
    )(word_emb_tab.astype(jnp.float32))

    labels3 = labels.reshape(nblk, 1, BLK)
    total = pl.pallas_call(
        _main_body,
        grid=(nblk,),
        in_specs=[
            pl.BlockSpec((BLK, V), lambda i: (i, 0)),
            pl.BlockSpec((1, 1, BLK), lambda i: (i, 0, 0)),
            pl.BlockSpec((V, V), lambda i: (0, 0)),
        ],
        out_specs=pl.BlockSpec(memory_space=pltpu.SMEM),
        out_shape=jax.ShapeDtypeStruct((1,), jnp.float32),
        scratch_shapes=[
            pltpu.SMEM((1,), jnp.float32),
        ],
    )(logits, labels3, s_tab)

    return (total[0] / batch).astype(jnp.float32)
